# Initial kernel scaffold; baseline (speedup 1.0000x reference)
import jax
import jax.numpy as jnp
from jax.experimental import pallas as pl


def kernel(inputs, pc_feat0, pc_feat1, pc_feat2, pc_feat3):
    raise NotImplementedError("write your pallas kernel here")


# speedup vs baseline: 30.1527x; 30.1527x over previous
import jax
import jax.numpy as jnp
from jax.experimental import pallas as pl

_K = 8
_ROW_TILE = 128


def _stage_body(x_ref, pc_ref, out_ref):
    x = x_ref[...]
    p = pc_ref[0]
    r, m = x.shape[0], p.shape[1]
    xp = jax.lax.dot_general(
        x, p, (((1,), (0,)), ((), ())),
        preferred_element_type=jnp.float32)
    x2 = jnp.sum(x * x, axis=1, keepdims=True)
    p2 = jnp.sum(p * p, axis=0, keepdims=True)
    d2 = (x2 + p2) - 2.0 * xp
    iota = jax.lax.broadcasted_iota(jnp.int32, (r, m), 1)
    acc = jnp.zeros((r, m), jnp.float32)
    for _ in range(_K):
        v = jnp.min(d2, axis=1, keepdims=True)
        idx = jnp.min(jnp.where(d2 == v, iota, m), axis=1, keepdims=True)
        onehot = iota == idx
        acc = jnp.where(onehot, 1.0, acc)
        d2 = jnp.where(onehot, jnp.inf, d2)
    mean = jax.lax.dot_general(
        acc, p, (((1,), (1,)), ((), ())),
        preferred_element_type=jnp.float32) * (1.0 / _K)
    out_ref[0] = mean


def kernel(inputs, pc_feat0, pc_feat1, pc_feat2, pc_feat3):
    n, d = inputs.shape
    pc_all = jnp.concatenate([pc_feat0, pc_feat1, pc_feat2, pc_feat3], axis=0)
    s, m = pc_all.shape[0], pc_all.shape[2]
    stages = pl.pallas_call(
        _stage_body,
        grid=(s, n // _ROW_TILE),
        in_specs=[
            pl.BlockSpec((_ROW_TILE, d), lambda si, i: (i, 0)),
            pl.BlockSpec((1, d, m), lambda si, i: (si, 0, 0)),
        ],
        out_specs=pl.BlockSpec((1, _ROW_TILE, d), lambda si, i: (si, i, 0)),
        out_shape=jax.ShapeDtypeStruct((s, n, d), jnp.float32),
    )(inputs, pc_all)
    return jnp.concatenate(
        [inputs, stages[0], stages[1], stages[2], stages[3]], axis=1)


# speedup vs baseline: 35.9552x; 1.1924x over previous
import jax
import jax.numpy as jnp
from jax import lax
from jax.experimental import pallas as pl
from jax.experimental.pallas import tpu as pltpu
from jax.experimental.pallas import tpu_sc as plsc

_K = 8
_ROW_TILE = 128
_GATHER_CHUNK = 128


def _topk_body(x_ref, pc_ref, idx_ref):
    x = x_ref[...]
    p = pc_ref[0]
    r, m = x.shape[0], p.shape[1]
    si = pl.program_id(0)
    xp = lax.dot_general(
        x, p, (((1,), (0,)), ((), ())),
        preferred_element_type=jnp.float32)
    x2 = jnp.sum(x * x, axis=1, keepdims=True)
    p2 = jnp.sum(p * p, axis=0, keepdims=True)
    d2 = (x2 + p2) - 2.0 * xp
    iota = lax.broadcasted_iota(jnp.int32, (r, m), 1)
    for k in range(_K):
        v = jnp.min(d2, axis=1, keepdims=True)
        idx = jnp.min(jnp.where(d2 == v, iota, m), axis=1, keepdims=True)
        idx_ref[0, :, k] = idx[:, 0] + si * m
        if k + 1 < _K:
            d2 = jnp.where(iota == idx, jnp.inf, d2)


def _topk_indices(inputs, pc_all):
    n, d = inputs.shape
    s, m = pc_all.shape[0], pc_all.shape[2]
    return pl.pallas_call(
        _topk_body,
        grid=(s, n // _ROW_TILE),
        in_specs=[
            pl.BlockSpec((_ROW_TILE, d), lambda si, i: (i, 0)),
            pl.BlockSpec((1, d, m), lambda si, i: (si, 0, 0)),
        ],
        out_specs=pl.BlockSpec((1, _ROW_TILE, _K), lambda si, i: (si, i, 0)),
        out_shape=jax.ShapeDtypeStruct((s, n, _K), jnp.int32),
    )(inputs, pc_all)


def _gather_mean(table, idx_flat, num_out_rows, d):
    info = plsc.get_sparse_core_info()
    nc, ns = info.num_cores, info.num_subcores
    nw = nc * ns
    g_per_w = idx_flat.shape[0] // nw
    chunks = g_per_w // _GATHER_CHUNK
    rows_per_chunk = _GATHER_CHUNK // _K
    idx3 = idx_flat.reshape(nw, chunks, _GATHER_CHUNK)
    mesh = plsc.VectorSubcoreMesh(core_axis_name="c", subcore_axis_name="s")

    def body(table_hbm, idx_hbm, out_hbm, idx_v, rows_v, out_v, sem):
        wid = lax.axis_index("s") * nc + lax.axis_index("c")
        pltpu.sync_copy(idx_hbm.at[wid], idx_v)

        def chunk_body(j, carry):
            pltpu.async_copy(
                table_hbm.at[idx_v.at[j]], rows_v, sem).wait()

            def row_body(o, c2):
                base = o * _K
                for cch in range(d // 16):
                    sl = pl.ds(cch * 16, 16)
                    a = rows_v[base, sl]
                    for nn in range(1, _K):
                        a = a + rows_v[base + nn, sl]
                    out_v[o, sl] = a * (1.0 / _K)
                return c2

            lax.fori_loop(0, rows_per_chunk, row_body, 0)
            pltpu.sync_copy(
                out_v,
                out_hbm.at[pl.ds(wid * (chunks * rows_per_chunk)
                                 + j * rows_per_chunk, rows_per_chunk)])
            return carry

        lax.fori_loop(0, chunks, chunk_body, 0)

    return pl.kernel(
        body,
        out_type=jax.ShapeDtypeStruct((num_out_rows, d), jnp.float32),
        mesh=mesh,
        compiler_params=pltpu.CompilerParams(use_tc_tiling_on_sc=False),
        scratch_types=[
            pltpu.VMEM((chunks, _GATHER_CHUNK), jnp.int32),
            pltpu.VMEM((_GATHER_CHUNK, d), jnp.float32),
            pltpu.VMEM((rows_per_chunk, d), jnp.float32),
            pltpu.SemaphoreType.DMA,
        ],
    )(table, idx3)


def kernel(inputs, pc_feat0, pc_feat1, pc_feat2, pc_feat3):
    n, d = inputs.shape
    pc_all = jnp.concatenate([pc_feat0, pc_feat1, pc_feat2, pc_feat3], axis=0)
    s, m = pc_all.shape[0], pc_all.shape[2]
    idx = _topk_indices(inputs, pc_all)
    table = jnp.transpose(pc_all, (0, 2, 1)).reshape(s * m, d)
    means = _gather_mean(table, idx.reshape(-1), s * n, d)
    stages = means.reshape(s, n, d)
    return jnp.concatenate(
        [inputs, stages[0], stages[1], stages[2], stages[3]], axis=1)
